# baseline (speedup 1.0000x reference)
import functools

import jax
import jax.numpy as jnp
from jax import lax
from jax.experimental import pallas as pl
from jax.experimental.pallas import tpu as pltpu
from jax.experimental.pallas import tpu_sc as plsc

DIM = 128
BRAND_DIM = 16
N_BRANDS = 1000
BATCH = 16384

NC = 2
NS = 16
NW = NC * NS
BPW = BATCH // NW
CHUNK = 128
K = BPW // CHUNK
G = BPW // 16

_SC_PARAMS = pltpu.CompilerParams(needs_layout_passes=False,
                                  use_tc_tiling_on_sc=False)


@functools.cache
def _make_sc_brand():
    mesh = plsc.VectorSubcoreMesh(core_axis_name="c", subcore_axis_name="s")

    @functools.partial(
        pl.kernel,
        mesh=mesh,
        compiler_params=_SC_PARAMS,
        out_type=jax.ShapeDtypeStruct((BATCH, BRAND_DIM), jnp.float32),
        scratch_types=[
            pltpu.VMEM((G, 16), jnp.int32),
            pltpu.VMEM((N_BRANDS * BRAND_DIM,), jnp.float32),
            pltpu.VMEM((BPW, BRAND_DIM), jnp.float32),
        ],
    )
    def _sc_brand(bidx_hbm, brands_hbm, bout, bidx_v, bt_v, bout_v):
        wid = lax.axis_index("s") * NC + lax.axis_index("c")
        base = wid * BPW
        pltpu.sync_copy(bidx_hbm.at[wid], bidx_v)
        pltpu.sync_copy(brands_hbm, bt_v)

        def brand_body(g, _):
            idx16 = bidx_v[g]
            flat0 = idx16 * BRAND_DIM
            rows16 = jnp.full((16,), g * 16, jnp.int32) + lax.iota(jnp.int32, 16)
            for j in range(BRAND_DIM):
                vals = plsc.load_gather(bt_v, [flat0 + j])
                plsc.store_scatter(bout_v, [rows16, jnp.full((16,), j, jnp.int32)],
                                   vals)
            return 0

        lax.fori_loop(0, G, brand_body, 0)
        pltpu.sync_copy(bout_v, bout.at[pl.ds(base, BPW)])

    return _sc_brand


@functools.cache
def _make_sc_gather():
    mesh = plsc.VectorSubcoreMesh(core_axis_name="c", subcore_axis_name="s")

    @functools.partial(
        pl.kernel,
        mesh=mesh,
        compiler_params=_SC_PARAMS,
        out_type=(
            jax.ShapeDtypeStruct((BATCH, DIM), jnp.float32),
            jax.ShapeDtypeStruct((BATCH, DIM), jnp.float32),
        ),
        scratch_types=[
            pltpu.VMEM((K, CHUNK), jnp.int32),
            pltpu.VMEM((K, CHUNK), jnp.int32),
            pltpu.VMEM((BPW, DIM), jnp.float32),
            pltpu.SemaphoreType.DMA,
            pltpu.SemaphoreType.DMA,
        ],
    )
    def _sc_gather(uidx_hbm, iidx_hbm, users_hbm, items_hbm,
                   uout, iout, uidx_v, iidx_v, rows_v, gsem, wsem):
        wid = lax.axis_index("s") * NC + lax.axis_index("c")
        base = wid * BPW
        pltpu.sync_copy(uidx_hbm.at[wid], uidx_v)
        pltpu.sync_copy(iidx_hbm.at[wid], iidx_v)

        ug = [pltpu.async_copy(users_hbm.at[uidx_v.at[j]],
                               rows_v.at[pl.ds(j * CHUNK, CHUNK)], gsem)
              for j in range(K)]
        uw = []
        for j in range(K):
            ug[j].wait()
            uw.append(pltpu.async_copy(rows_v.at[pl.ds(j * CHUNK, CHUNK)],
                                       uout.at[pl.ds(base + j * CHUNK, CHUNK)],
                                       wsem))
        ig = []
        for j in range(K):
            uw[j].wait()
            ig.append(pltpu.async_copy(items_hbm.at[iidx_v.at[j]],
                                       rows_v.at[pl.ds(j * CHUNK, CHUNK)], gsem))
        iw = []
        for j in range(K):
            ig[j].wait()
            iw.append(pltpu.async_copy(rows_v.at[pl.ds(j * CHUNK, CHUNK)],
                                       iout.at[pl.ds(base + j * CHUNK, CHUNK)],
                                       wsem))
        for j in range(K):
            iw[j].wait()

    return _sc_gather


BM = 2048
XCOLS = 31
FF1 = 299


def _mlp_body(u, it, br, x, w1, b1, w2, b2, w3, b3, o):
    f32 = jnp.float32
    h = jnp.dot(u[:], w1[0:DIM, :], preferred_element_type=f32)
    h = h + jnp.dot(it[:], w1[DIM:2 * DIM, :], preferred_element_type=f32)
    h = h + jnp.dot(br[:], w1[256:272, :], preferred_element_type=f32)
    h = h + jnp.dot(x[:, 2:18], w1[272:288, :], preferred_element_type=f32)
    h = h + jnp.dot(x[:, 20:31], w1[288:299, :], preferred_element_type=f32)
    h = jnp.maximum(h + b1[:], 0.0)
    h2 = jnp.maximum(jnp.dot(h, w2[:], preferred_element_type=f32) + b2[:], 0.0)
    s = jnp.sum(h2 * w3[:], axis=1) + b3[0, 0]
    o[:] = jnp.tanh(s)


def _mlp(u, it, br, x, w1, b1, w2, b2, w3, b3):
    grid = (BATCH // BM,)
    row = lambda i: (i, 0)
    rep = lambda i: (0, 0)
    return pl.pallas_call(
        _mlp_body,
        grid=grid,
        in_specs=[
            pl.BlockSpec((BM, DIM), row),
            pl.BlockSpec((BM, DIM), row),
            pl.BlockSpec((BM, BRAND_DIM), row),
            pl.BlockSpec((BM, XCOLS), row),
            pl.BlockSpec((FF1, DIM), rep),
            pl.BlockSpec((1, DIM), rep),
            pl.BlockSpec((DIM, 32), rep),
            pl.BlockSpec((1, 32), rep),
            pl.BlockSpec((1, 32), rep),
            pl.BlockSpec((1, 1), rep),
        ],
        out_specs=pl.BlockSpec((BM,), lambda i: (i,)),
        out_shape=jax.ShapeDtypeStruct((BATCH,), jnp.float32),
    )(u, it, br, x, w1, b1, w2, b2, w3, b3)


def kernel(x, users_emb, items_emb, brands_emb, W1, b1, W2, b2, W3, b3):
    uidx = x[:, 0].astype(jnp.int32).reshape(NW, K, CHUNK)
    iidx = x[:, 1].astype(jnp.int32).reshape(NW, K, CHUNK)
    bidx = x[:, 19].astype(jnp.int32).reshape(NW, G, 16)

    b_g = _make_sc_brand()(bidx, brands_emb.reshape(-1))
    u_g, i_g = _make_sc_gather()(uidx, iidx, users_emb, items_emb)

    out = _mlp(u_g, i_g, b_g, x, W1,
               b1.reshape(1, DIM), W2, b2.reshape(1, 32),
               W3.reshape(1, 32), b3.reshape(1, 1))
    return out.reshape(BATCH, 1)


# speedup vs baseline: 1.0002x; 1.0002x over previous
import functools

import jax
import jax.numpy as jnp
from jax import lax
from jax.experimental import pallas as pl
from jax.experimental.pallas import tpu as pltpu
from jax.experimental.pallas import tpu_sc as plsc

DIM = 128
BRAND_DIM = 16
N_BRANDS = 1000
BATCH = 16384

NC = 2
NS = 16
NW = NC * NS
BPW = BATCH // NW
CHUNK = 128
K = BPW // CHUNK
G = BPW // 16

_SC_PARAMS = pltpu.CompilerParams(needs_layout_passes=False,
                                  use_tc_tiling_on_sc=False)


@functools.cache
def _make_sc_brand():
    mesh = plsc.VectorSubcoreMesh(core_axis_name="c", subcore_axis_name="s")

    @functools.partial(
        pl.kernel,
        mesh=mesh,
        compiler_params=_SC_PARAMS,
        out_type=jax.ShapeDtypeStruct((BATCH, BRAND_DIM), jnp.float32),
        scratch_types=[
            pltpu.VMEM((G, 16), jnp.int32),
            pltpu.VMEM((N_BRANDS * BRAND_DIM,), jnp.float32),
            pltpu.VMEM((BPW, BRAND_DIM), jnp.float32),
        ],
    )
    def _sc_brand(bidx_hbm, brands_hbm, bout, bidx_v, bt_v, bout_v):
        wid = lax.axis_index("s") * NC + lax.axis_index("c")
        base = wid * BPW
        pltpu.sync_copy(bidx_hbm.at[wid], bidx_v)
        pltpu.sync_copy(brands_hbm, bt_v)

        def brand_body(g, _):
            idx16 = bidx_v[g]
            flat0 = idx16 * BRAND_DIM
            rows16 = jnp.full((16,), g * 16, jnp.int32) + lax.iota(jnp.int32, 16)
            for j in range(BRAND_DIM):
                vals = plsc.load_gather(bt_v, [flat0 + j])
                plsc.store_scatter(bout_v, [rows16, jnp.full((16,), j, jnp.int32)],
                                   vals)
            return 0

        lax.fori_loop(0, G, brand_body, 0)
        pltpu.sync_copy(bout_v, bout.at[pl.ds(base, BPW)])

    return _sc_brand


@functools.cache
def _make_sc_gather():
    mesh = plsc.VectorSubcoreMesh(core_axis_name="c", subcore_axis_name="s")

    @functools.partial(
        pl.kernel,
        mesh=mesh,
        compiler_params=_SC_PARAMS,
        out_type=(
            jax.ShapeDtypeStruct((BATCH, DIM), jnp.float32),
            jax.ShapeDtypeStruct((BATCH, DIM), jnp.float32),
        ),
        scratch_types=[
            pltpu.VMEM((K, CHUNK), jnp.int32),
            pltpu.VMEM((K, CHUNK), jnp.int32),
            pltpu.VMEM((BPW, DIM), jnp.float32),
            pltpu.SemaphoreType.DMA,
            pltpu.SemaphoreType.DMA,
        ],
    )
    def _sc_gather(uidx_hbm, iidx_hbm, users_hbm, items_hbm,
                   uout, iout, uidx_v, iidx_v, rows_v, gsem, wsem):
        wid = lax.axis_index("s") * NC + lax.axis_index("c")
        base = wid * BPW
        pltpu.sync_copy(uidx_hbm.at[wid], uidx_v)
        pltpu.sync_copy(iidx_hbm.at[wid], iidx_v)

        ug = [pltpu.async_copy(users_hbm.at[uidx_v.at[j]],
                               rows_v.at[pl.ds(j * CHUNK, CHUNK)], gsem)
              for j in range(K)]
        uw = []
        for j in range(K):
            ug[j].wait()
            uw.append(pltpu.async_copy(rows_v.at[pl.ds(j * CHUNK, CHUNK)],
                                       uout.at[pl.ds(base + j * CHUNK, CHUNK)],
                                       wsem))
        ig = []
        for j in range(K):
            uw[j].wait()
            ig.append(pltpu.async_copy(items_hbm.at[iidx_v.at[j]],
                                       rows_v.at[pl.ds(j * CHUNK, CHUNK)], gsem))
        iw = []
        for j in range(K):
            ig[j].wait()
            iw.append(pltpu.async_copy(rows_v.at[pl.ds(j * CHUNK, CHUNK)],
                                       iout.at[pl.ds(base + j * CHUNK, CHUNK)],
                                       wsem))
        for j in range(K):
            iw[j].wait()

    return _sc_gather


BM = 2048
XCOLS = 31
FF1 = 299


def _mlp_body(u, it, br, x, w1, b1, w2, b2, w3, b3, o, w1x):
    f32 = jnp.float32

    @pl.when(pl.program_id(0) == 0)
    def _():
        w1x[...] = jnp.zeros((32, DIM), f32)
        w1x[2:18, :] = w1[272:288, :]
        w1x[20:31, :] = w1[288:299, :]

    h = jnp.dot(u[:], w1[0:DIM, :], preferred_element_type=f32)
    h = h + jnp.dot(it[:], w1[DIM:2 * DIM, :], preferred_element_type=f32)
    h = h + jnp.dot(br[:], w1[256:272, :], preferred_element_type=f32)
    h = h + jnp.dot(x[:], w1x[0:XCOLS, :], preferred_element_type=f32)
    h = jnp.maximum(h + b1[:], 0.0)
    h2 = jnp.maximum(jnp.dot(h, w2[:], preferred_element_type=f32) + b2[:], 0.0)
    s = jnp.sum(h2 * w3[:], axis=1) + b3[0, 0]
    o[:] = jnp.tanh(s)


def _mlp(u, it, br, x, w1, b1, w2, b2, w3, b3):
    grid = (BATCH // BM,)
    row = lambda i: (i, 0)
    rep = lambda i: (0, 0)
    return pl.pallas_call(
        _mlp_body,
        grid=grid,
        in_specs=[
            pl.BlockSpec((BM, DIM), row),
            pl.BlockSpec((BM, DIM), row),
            pl.BlockSpec((BM, BRAND_DIM), row),
            pl.BlockSpec((BM, XCOLS), row),
            pl.BlockSpec((FF1, DIM), rep),
            pl.BlockSpec((1, DIM), rep),
            pl.BlockSpec((DIM, 32), rep),
            pl.BlockSpec((1, 32), rep),
            pl.BlockSpec((1, 32), rep),
            pl.BlockSpec((1, 1), rep),
        ],
        out_specs=pl.BlockSpec((BM,), lambda i: (i,)),
        out_shape=jax.ShapeDtypeStruct((BATCH,), jnp.float32),
        scratch_shapes=[pltpu.VMEM((32, DIM), jnp.float32)],
    )(u, it, br, x, w1, b1, w2, b2, w3, b3)


def kernel(x, users_emb, items_emb, brands_emb, W1, b1, W2, b2, W3, b3):
    uidx = x[:, 0].astype(jnp.int32).reshape(NW, K, CHUNK)
    iidx = x[:, 1].astype(jnp.int32).reshape(NW, K, CHUNK)
    bidx = x[:, 19].astype(jnp.int32).reshape(NW, G, 16)

    b_g = _make_sc_brand()(bidx, brands_emb.reshape(-1))
    u_g, i_g = _make_sc_gather()(uidx, iidx, users_emb, items_emb)

    out = _mlp(u_g, i_g, b_g, x, W1,
               b1.reshape(1, DIM), W2, b2.reshape(1, 32),
               W3.reshape(1, 32), b3.reshape(1, 1))
    return out.reshape(BATCH, 1)


# speedup vs baseline: 1.1993x; 1.1991x over previous
import functools

import jax
import jax.numpy as jnp
from jax import lax
from jax.experimental import pallas as pl
from jax.experimental.pallas import tpu as pltpu
from jax.experimental.pallas import tpu_sc as plsc

DIM = 128
BRAND_DIM = 16
N_BRANDS = 1000
BATCH = 16384

NC = 2
NS = 16
NW = NC * NS
BPW = BATCH // NW
CHUNK = 128
K = BPW // CHUNK
G = BPW // 16

_SC_PARAMS = pltpu.CompilerParams(needs_layout_passes=False,
                                  use_tc_tiling_on_sc=False)


@functools.cache
def _make_sc_brand():
    mesh = plsc.VectorSubcoreMesh(core_axis_name="c", subcore_axis_name="s")

    @functools.partial(
        pl.kernel,
        mesh=mesh,
        compiler_params=_SC_PARAMS,
        out_type=jax.ShapeDtypeStruct((BATCH, BRAND_DIM), jnp.float32),
        scratch_types=[
            pltpu.VMEM((K, CHUNK), jnp.int32),
            pltpu.VMEM((BPW, BRAND_DIM), jnp.float32),
            pltpu.SemaphoreType.DMA,
        ],
    )
    def _sc_brand(bidx_hbm, brands_hbm, bout, bidx_v, bout_v, sem):
        wid = lax.axis_index("s") * NC + lax.axis_index("c")
        base = wid * BPW
        pltpu.sync_copy(bidx_hbm.at[wid], bidx_v)
        copies = [
            pltpu.async_copy(brands_hbm.at[bidx_v.at[j]],
                             bout_v.at[pl.ds(j * CHUNK, CHUNK)], sem)
            for j in range(K)
        ]
        for c in copies:
            c.wait()
        pltpu.sync_copy(bout_v, bout.at[pl.ds(base, BPW)])

    return _sc_brand


@functools.cache
def _make_sc_gather():
    mesh = plsc.VectorSubcoreMesh(core_axis_name="c", subcore_axis_name="s")

    @functools.partial(
        pl.kernel,
        mesh=mesh,
        compiler_params=_SC_PARAMS,
        out_type=(
            jax.ShapeDtypeStruct((BATCH, DIM), jnp.float32),
            jax.ShapeDtypeStruct((BATCH, DIM), jnp.float32),
        ),
        scratch_types=[
            pltpu.VMEM((K, CHUNK), jnp.int32),
            pltpu.VMEM((K, CHUNK), jnp.int32),
            pltpu.VMEM((BPW, DIM), jnp.float32),
            pltpu.SemaphoreType.DMA,
            pltpu.SemaphoreType.DMA,
        ],
    )
    def _sc_gather(uidx_hbm, iidx_hbm, users_hbm, items_hbm,
                   uout, iout, uidx_v, iidx_v, rows_v, gsem, wsem):
        wid = lax.axis_index("s") * NC + lax.axis_index("c")
        base = wid * BPW
        pltpu.sync_copy(uidx_hbm.at[wid], uidx_v)
        pltpu.sync_copy(iidx_hbm.at[wid], iidx_v)

        ug = [pltpu.async_copy(users_hbm.at[uidx_v.at[j]],
                               rows_v.at[pl.ds(j * CHUNK, CHUNK)], gsem)
              for j in range(K)]
        uw = []
        for j in range(K):
            ug[j].wait()
            uw.append(pltpu.async_copy(rows_v.at[pl.ds(j * CHUNK, CHUNK)],
                                       uout.at[pl.ds(base + j * CHUNK, CHUNK)],
                                       wsem))
        ig = []
        for j in range(K):
            uw[j].wait()
            ig.append(pltpu.async_copy(items_hbm.at[iidx_v.at[j]],
                                       rows_v.at[pl.ds(j * CHUNK, CHUNK)], gsem))
        iw = []
        for j in range(K):
            ig[j].wait()
            iw.append(pltpu.async_copy(rows_v.at[pl.ds(j * CHUNK, CHUNK)],
                                       iout.at[pl.ds(base + j * CHUNK, CHUNK)],
                                       wsem))
        for j in range(K):
            iw[j].wait()

    return _sc_gather


BM = 2048
XCOLS = 31
FF1 = 299


def _mlp_body(u, it, br, x, w1, b1, w2, b2, w3, b3, o, w1x):
    f32 = jnp.float32

    @pl.when(pl.program_id(0) == 0)
    def _():
        w1x[...] = jnp.zeros((32, DIM), f32)
        w1x[2:18, :] = w1[272:288, :]
        w1x[20:31, :] = w1[288:299, :]

    h = jnp.dot(u[:], w1[0:DIM, :], preferred_element_type=f32)
    h = h + jnp.dot(it[:], w1[DIM:2 * DIM, :], preferred_element_type=f32)
    h = h + jnp.dot(br[:], w1[256:272, :], preferred_element_type=f32)
    h = h + jnp.dot(x[:], w1x[0:XCOLS, :], preferred_element_type=f32)
    h = jnp.maximum(h + b1[:], 0.0)
    h2 = jnp.maximum(jnp.dot(h, w2[:], preferred_element_type=f32) + b2[:], 0.0)
    o[:] = jnp.tanh(jnp.dot(h2, w3[:], preferred_element_type=f32) + b3[:])


def _mlp(u, it, br, x, w1, b1, w2, b2, w3, b3):
    grid = (BATCH // BM,)
    row = lambda i: (i, 0)
    rep = lambda i: (0, 0)
    return pl.pallas_call(
        _mlp_body,
        grid=grid,
        in_specs=[
            pl.BlockSpec((BM, DIM), row),
            pl.BlockSpec((BM, DIM), row),
            pl.BlockSpec((BM, BRAND_DIM), row),
            pl.BlockSpec((BM, XCOLS), row),
            pl.BlockSpec((FF1, DIM), rep),
            pl.BlockSpec((1, DIM), rep),
            pl.BlockSpec((DIM, 32), rep),
            pl.BlockSpec((1, 32), rep),
            pl.BlockSpec((32, 1), rep),
            pl.BlockSpec((1, 1), rep),
        ],
        out_specs=pl.BlockSpec((BM, 1), row),
        out_shape=jax.ShapeDtypeStruct((BATCH, 1), jnp.float32),
        scratch_shapes=[pltpu.VMEM((32, DIM), jnp.float32)],
    )(u, it, br, x, w1, b1, w2, b2, w3, b3)


def kernel(x, users_emb, items_emb, brands_emb, W1, b1, W2, b2, W3, b3):
    uidx = x[:, 0].astype(jnp.int32).reshape(NW, K, CHUNK)
    iidx = x[:, 1].astype(jnp.int32).reshape(NW, K, CHUNK)
    bidx = x[:, 19].astype(jnp.int32).reshape(NW, K, CHUNK)

    b_g = _make_sc_brand()(bidx, brands_emb)
    u_g, i_g = _make_sc_gather()(uidx, iidx, users_emb, items_emb)

    return _mlp(u_g, i_g, b_g, x, W1,
                b1.reshape(1, DIM), W2, b2.reshape(1, 32),
                W3, b3.reshape(1, 1))


# speedup vs baseline: 1.2098x; 1.0087x over previous
import functools

import jax
import jax.numpy as jnp
from jax import lax
from jax.experimental import pallas as pl
from jax.experimental.pallas import tpu as pltpu
from jax.experimental.pallas import tpu_sc as plsc

DIM = 128
BRAND_DIM = 16
N_BRANDS = 1000
BATCH = 16384

NC = 2
NS = 16
NW = NC * NS
BPW = BATCH // NW
CHUNK = 128
K = BPW // CHUNK
G = BPW // 16

_SC_PARAMS = pltpu.CompilerParams(needs_layout_passes=False,
                                  use_tc_tiling_on_sc=False)


@functools.cache
def _make_sc_brand():
    mesh = plsc.VectorSubcoreMesh(core_axis_name="c", subcore_axis_name="s")

    @functools.partial(
        pl.kernel,
        mesh=mesh,
        compiler_params=_SC_PARAMS,
        out_type=jax.ShapeDtypeStruct((BATCH, BRAND_DIM), jnp.float32),
        scratch_types=[
            pltpu.VMEM((K, CHUNK), jnp.int32),
            pltpu.VMEM((BPW, BRAND_DIM), jnp.float32),
            pltpu.SemaphoreType.DMA,
        ],
    )
    def _sc_brand(bidx_hbm, brands_hbm, bout, bidx_v, bout_v, sem):
        wid = lax.axis_index("s") * NC + lax.axis_index("c")
        base = wid * BPW
        pltpu.sync_copy(bidx_hbm.at[wid], bidx_v)
        copies = [
            pltpu.async_copy(brands_hbm.at[bidx_v.at[j]],
                             bout_v.at[pl.ds(j * CHUNK, CHUNK)], sem)
            for j in range(K)
        ]
        for c in copies:
            c.wait()
        pltpu.sync_copy(bout_v, bout.at[pl.ds(base, BPW)])

    return _sc_brand


@functools.cache
def _make_sc_gather():
    mesh = plsc.VectorSubcoreMesh(core_axis_name="c", subcore_axis_name="s")

    @functools.partial(
        pl.kernel,
        mesh=mesh,
        compiler_params=_SC_PARAMS,
        out_type=(
            jax.ShapeDtypeStruct((BATCH, DIM), jnp.float32),
            jax.ShapeDtypeStruct((BATCH, DIM), jnp.float32),
        ),
        scratch_types=[
            pltpu.VMEM((K, CHUNK), jnp.int32),
            pltpu.VMEM((K, CHUNK), jnp.int32),
            pltpu.VMEM((BPW, DIM), jnp.float32),
            pltpu.SemaphoreType.DMA,
            pltpu.SemaphoreType.DMA,
        ],
    )
    def _sc_gather(uidx_hbm, iidx_hbm, users_hbm, items_hbm,
                   uout, iout, uidx_v, iidx_v, rows_v, gsem, wsem):
        wid = lax.axis_index("s") * NC + lax.axis_index("c")
        base = wid * BPW
        pltpu.sync_copy(uidx_hbm.at[wid], uidx_v)
        pltpu.sync_copy(iidx_hbm.at[wid], iidx_v)

        ug = [pltpu.async_copy(users_hbm.at[uidx_v.at[j]],
                               rows_v.at[pl.ds(j * CHUNK, CHUNK)], gsem)
              for j in range(K)]
        uw = []
        for j in range(K):
            ug[j].wait()
            uw.append(pltpu.async_copy(rows_v.at[pl.ds(j * CHUNK, CHUNK)],
                                       uout.at[pl.ds(base + j * CHUNK, CHUNK)],
                                       wsem))
        ig = []
        for j in range(K):
            uw[j].wait()
            ig.append(pltpu.async_copy(items_hbm.at[iidx_v.at[j]],
                                       rows_v.at[pl.ds(j * CHUNK, CHUNK)], gsem))
        iw = []
        for j in range(K):
            ig[j].wait()
            iw.append(pltpu.async_copy(rows_v.at[pl.ds(j * CHUNK, CHUNK)],
                                       iout.at[pl.ds(base + j * CHUNK, CHUNK)],
                                       wsem))
        for j in range(K):
            iw[j].wait()

    return _sc_gather


BM = 4096
XCOLS = 31
FF1 = 299


def _mlp_body(u, it, br, x, w1, b1, w2, b2, w3, b3, o, w1x):
    f32 = jnp.float32

    @pl.when(pl.program_id(0) == 0)
    def _():
        w1x[...] = jnp.zeros((32, DIM), f32)
        w1x[2:18, :] = w1[272:288, :]
        w1x[20:31, :] = w1[288:299, :]

    h = jnp.dot(u[:], w1[0:DIM, :], preferred_element_type=f32)
    h = h + jnp.dot(it[:], w1[DIM:2 * DIM, :], preferred_element_type=f32)
    h = h + jnp.dot(br[:], w1[256:272, :], preferred_element_type=f32)
    h = h + jnp.dot(x[:], w1x[0:XCOLS, :], preferred_element_type=f32)
    h = jnp.maximum(h + b1[:], 0.0)
    h2 = jnp.maximum(jnp.dot(h, w2[:], preferred_element_type=f32) + b2[:], 0.0)
    o[:] = jnp.tanh(jnp.dot(h2, w3[:], preferred_element_type=f32) + b3[:])


def _mlp(u, it, br, x, w1, b1, w2, b2, w3, b3):
    grid = (BATCH // BM,)
    row = lambda i: (i, 0)
    rep = lambda i: (0, 0)
    return pl.pallas_call(
        _mlp_body,
        grid=grid,
        in_specs=[
            pl.BlockSpec((BM, DIM), row),
            pl.BlockSpec((BM, DIM), row),
            pl.BlockSpec((BM, BRAND_DIM), row),
            pl.BlockSpec((BM, XCOLS), row),
            pl.BlockSpec((FF1, DIM), rep),
            pl.BlockSpec((1, DIM), rep),
            pl.BlockSpec((DIM, 32), rep),
            pl.BlockSpec((1, 32), rep),
            pl.BlockSpec((32, 1), rep),
            pl.BlockSpec((1, 1), rep),
        ],
        out_specs=pl.BlockSpec((BM, 1), row),
        out_shape=jax.ShapeDtypeStruct((BATCH, 1), jnp.float32),
        scratch_shapes=[pltpu.VMEM((32, DIM), jnp.float32)],
    )(u, it, br, x, w1, b1, w2, b2, w3, b3)


def kernel(x, users_emb, items_emb, brands_emb, W1, b1, W2, b2, W3, b3):
    uidx = x[:, 0].astype(jnp.int32).reshape(NW, K, CHUNK)
    iidx = x[:, 1].astype(jnp.int32).reshape(NW, K, CHUNK)
    bidx = x[:, 19].astype(jnp.int32).reshape(NW, K, CHUNK)

    b_g = _make_sc_brand()(bidx, brands_emb)
    u_g, i_g = _make_sc_gather()(uidx, iidx, users_emb, items_emb)

    return _mlp(u_g, i_g, b_g, x, W1,
                b1.reshape(1, DIM), W2, b2.reshape(1, 32),
                W3, b3.reshape(1, 1))
